# baseline (speedup 1.0000x reference)
import functools

import jax
import jax.numpy as jnp
from jax.experimental import pallas as pl
from jax.experimental.pallas import tpu as pltpu

B, T = 4096, 50
D_IN, ENC, HID = 11, 128, 128
STEPS = 50
BB = 512


def _fused_kernel(x_ref, w1_ref, b1_ref, w2_ref, b2_ref, wc_ref, bc_ref,
                  cw1_ref, cb1_ref, cw2_ref, cb2_ref,
                  lw1_ref, lb1_ref, lw2_ref, lb2_ref,
                  coords_ref, labels_ref, z_scr):
    f32 = jnp.float32
    x = x_ref[...].reshape(T * BB, D_IN)
    z = jnp.maximum(jnp.dot(x, w1_ref[...], preferred_element_type=f32)
                    + b1_ref[...], 0.0)
    z = jnp.maximum(jnp.dot(z, w2_ref[...], preferred_element_type=f32)
                    + b2_ref[...], 0.0)
    z_scr[...] = z.reshape(T, BB, ENC)

    wc = wc_ref[...]
    bc = bc_ref[...]

    def step(t, carry):
        h, c = carry
        zh = jnp.concatenate([z_scr[t], h], axis=1)
        gates = jnp.dot(zh, wc, preferred_element_type=f32) + bc
        i_t = jax.nn.sigmoid(gates[:, 0 * HID:1 * HID])
        f_t = jax.nn.sigmoid(gates[:, 1 * HID:2 * HID])
        g_t = jnp.tanh(gates[:, 2 * HID:3 * HID])
        o_t = jax.nn.sigmoid(gates[:, 3 * HID:4 * HID])
        c_new = f_t * c + i_t * g_t
        h_new = o_t * jnp.tanh(c_new)
        return h_new, c_new

    h0 = jnp.zeros((BB, HID), dtype=f32)
    c0 = jnp.zeros((BB, HID), dtype=f32)
    h, _ = jax.lax.fori_loop(0, T, step, (h0, c0))

    hc = jnp.maximum(jnp.dot(h, cw1_ref[...], preferred_element_type=f32)
                     + cb1_ref[...], 0.0)
    coords_ref[...] = jnp.dot(hc, cw2_ref[...], preferred_element_type=f32) + cb2_ref[...]
    hl = jnp.maximum(jnp.dot(h, lw1_ref[...], preferred_element_type=f32)
                     + lb1_ref[...], 0.0)
    labels_ref[...] = jnp.dot(hl, lw2_ref[...], preferred_element_type=f32) + lb2_ref[...]


def _full(shape):
    return pl.BlockSpec(shape, lambda i: (0,) * len(shape))


@functools.partial(jax.jit, static_argnames=("interpret",))
def _run(x, enc_W1, enc_b1, enc_W2, enc_b2, W_cat, b_cat,
         coord_W1, coord_b1, coord_W2, coord_b2,
         lab_W1, lab_b1, lab_W2, lab_b2, interpret=False):
    n_tiles = B // BB
    out_shapes = (
        jax.ShapeDtypeStruct((B, 3 * STEPS), jnp.float32),
        jax.ShapeDtypeStruct((B, STEPS), jnp.float32),
    )
    return pl.pallas_call(
        _fused_kernel,
        grid=(n_tiles,),
        in_specs=[
            pl.BlockSpec((T, BB, D_IN), lambda i: (0, i, 0)),
            _full((D_IN, ENC)), _full((1, ENC)),
            _full((ENC, ENC)), _full((1, ENC)),
            _full((ENC + HID, 4 * HID)), _full((1, 4 * HID)),
            _full((HID, HID)), _full((1, HID)),
            _full((HID, 3 * STEPS)), _full((1, 3 * STEPS)),
            _full((HID, HID // 2)), _full((1, HID // 2)),
            _full((HID // 2, STEPS)), _full((1, STEPS)),
        ],
        out_specs=(
            pl.BlockSpec((BB, 3 * STEPS), lambda i: (i, 0)),
            pl.BlockSpec((BB, STEPS), lambda i: (i, 0)),
        ),
        out_shape=out_shapes,
        scratch_shapes=[pltpu.VMEM((T, BB, ENC), jnp.float32)],
        compiler_params=pltpu.CompilerParams(
            dimension_semantics=("parallel",),
        ),
        interpret=interpret,
    )(x, enc_W1, enc_b1, enc_W2, enc_b2, W_cat, b_cat,
      coord_W1, coord_b1, coord_W2, coord_b2,
      lab_W1, lab_b1, lab_W2, lab_b2)


def kernel(obs_l, obs_c, obs_m, enc_W1, enc_b1, enc_W2, enc_b2,
           W_ih, W_hh, b_ih, b_hh,
           coord_W1, coord_b1, coord_W2, coord_b2,
           lab_W1, lab_b1, lab_W2, lab_b2):
    x = jnp.concatenate([obs_l, obs_c, obs_m], axis=-1)
    x = jnp.swapaxes(x, 0, 1)
    W_cat = jnp.concatenate([W_ih, W_hh], axis=0)
    b_cat = (b_ih + b_hh).reshape(1, 4 * HID)
    return _run(x, enc_W1, enc_b1.reshape(1, ENC), enc_W2, enc_b2.reshape(1, ENC),
                W_cat, b_cat,
                coord_W1, coord_b1.reshape(1, HID), coord_W2, coord_b2.reshape(1, 3 * STEPS),
                lab_W1, lab_b1.reshape(1, HID // 2), lab_W2, lab_b2.reshape(1, STEPS))
